# baseline (speedup 1.0000x reference)
import functools
from typing import Sequence

import jax
import jax.numpy as jnp
from jax import lax
from jax.experimental import pallas as pl
from jax.experimental.pallas import tpu as pltpu
from jax.experimental.pallas import tpu_sc as plsc

_NODE_TYPES = ['Path', 'DNS_Host', 'Package_Name', 'IP', 'Hostnames', 'Command', 'Port']
_EDGE_TYPES = [('Path', 'accesses', 'DNS_Host'), ('DNS_Host', 'resolves', 'IP'),
               ('Package_Name', 'installs', 'Path'), ('IP', 'listens', 'Port'),
               ('Hostnames', 'maps', 'IP'), ('Command', 'writes', 'Path')]
_N, _E, _D = 10000, 50000, 256
_NS = 2
_DS = _D // _NS
_R = 10112
_PAD_ROW = _N
_C = 128

_PAIRS = [(0, 0), (0, 2), (0, 5), (1, 0), (1, 1), (2, 2),
          (3, 1), (3, 3), (3, 4), (4, 4), (5, 5), (6, 3)]
_NREL = [3, 2, 1, 3, 1, 1, 1]
_CNT_PAIR = {0: 0, 2: 1, 5: 2, 1: 4, 3: 7, 4: 8}


def _sc_segment_sums(x14, src, dst, e_pad):
    mesh = plsc.VectorSubcoreMesh(core_axis_name="c", subcore_axis_name="s")
    n_cores = mesh.num_cores
    n_sub = mesh.num_subcores
    ep_sub = e_pad // n_sub
    n_chunks = ep_sub // _C
    assert n_chunks % 2 == 0 and n_chunks >= 4
    slab = _R // n_sub

    tasks = []
    for p, (i, r) in enumerate(_PAIRS):
        for h in range(_NS):
            tasks.append(('agg', i * _NS + h, r, p * _NS + h))
    for r in range(6):
        tasks.append(('cnt', None, r, r))
    n_slots = (len(tasks) + n_cores - 1) // n_cores
    by_core = [[None] * n_slots for _ in range(n_cores)]
    for j, t in enumerate(tasks):
        by_core[j % n_cores][j // n_cores] = t

    def body(x_hbm, src_hbm, dst_hbm, zer_hbm, ones_hbm, agg_hbm, cnt_hbm,
             src_a, dst_a, rows_a, src_b, dst_b, rows_b, acc,
             sem_a, sem_b, sem_sa, sem_sb):
        cid = lax.axis_index("c")
        sid = lax.axis_index("s")
        base_e = sid * ep_sub
        base_r = sid * slab

        def _gather(q, r, k, src_v, rows_v, sem):
            off = base_e + k * _C
            pltpu.sync_copy(src_hbm.at[r, pl.ds(off, _C)], src_v)
            return pltpu.async_copy(x_hbm.at[q].at[src_v], rows_v, sem)

        def _load_dst(r, k, dst_v):
            off = base_e + k * _C
            pltpu.sync_copy(dst_hbm.at[r, pl.ds(off, _C)], dst_v)

        def _scatter(r, k, dst_v, rows_v):
            _load_dst(r, k, dst_v)
            pltpu.sync_copy(rows_v, acc.at[dst_v], add=True)

        for s in range(n_slots):
            for c in range(n_cores):
                task = by_core[c][s]
                if task is None:
                    continue
                kind = task[0]

                @pl.when(cid == c)
                def _zero():
                    pltpu.sync_copy(zer_hbm, acc.at[pl.ds(base_r, slab)])
                    if kind == 'cnt':
                        pltpu.sync_copy(ones_hbm, rows_a)

            plsc.subcore_barrier()

            for c in range(n_cores):
                task = by_core[c][s]
                if task is None:
                    continue
                kind, q, r, g = task

                @pl.when(cid == c)
                def _accumulate():
                    if kind == 'cnt':
                        def _chunk(k, carry):
                            _scatter(r, k, dst_a, rows_a)
                            return carry

                        lax.fori_loop(0, n_chunks, _chunk, 0)
                    else:
                        ga = _gather(q, r, 0, src_a, rows_a, sem_a)

                        def _chunk2(k2, carry):
                            k = 2 * k2
                            gb = _gather(q, r, k + 1, src_b, rows_b, sem_b)
                            _load_dst(r, k, dst_a)
                            pltpu.make_async_copy(
                                x_hbm.at[q].at[src_a], rows_a, sem_a).wait()
                            sa = pltpu.async_copy(rows_a, acc.at[dst_a],
                                                  sem_sa, add=True)
                            _load_dst(r, k + 1, dst_b)
                            gb.wait()
                            sb = pltpu.async_copy(rows_b, acc.at[dst_b],
                                                  sem_sb, add=True)
                            sa.wait()
                            ga2 = _gather(q, r, k + 2, src_a, rows_a, sem_a)
                            sb.wait()
                            return carry

                        lax.fori_loop(0, n_chunks // 2 - 1, _chunk2, 0)
                        k_last = n_chunks - 2
                        gb = _gather(q, r, k_last + 1, src_b, rows_b, sem_b)
                        _load_dst(r, k_last, dst_a)
                        pltpu.make_async_copy(
                            x_hbm.at[q].at[src_a], rows_a, sem_a).wait()
                        sa = pltpu.async_copy(rows_a, acc.at[dst_a],
                                              sem_sa, add=True)
                        _load_dst(r, k_last + 1, dst_b)
                        gb.wait()
                        sb = pltpu.async_copy(rows_b, acc.at[dst_b],
                                              sem_sb, add=True)
                        sa.wait()
                        sb.wait()

            plsc.subcore_barrier()

            for c in range(n_cores):
                task = by_core[c][s]
                if task is None:
                    continue
                kind, q, r, g = task

                @pl.when(cid == c)
                def _drain():
                    out = agg_hbm if kind == 'agg' else cnt_hbm
                    pltpu.sync_copy(acc.at[pl.ds(base_r, slab)],
                                    out.at[g, pl.ds(base_r, slab)])

    zer = jnp.zeros((slab, _DS), jnp.float32)
    ones = jnp.ones((_C, _DS), jnp.float32)

    f = pl.kernel(
        body,
        out_type=(jax.ShapeDtypeStruct((12 * _NS, _R, _DS), jnp.float32),
                  jax.ShapeDtypeStruct((6, _R, _DS), jnp.float32)),
        mesh=mesh,
        scratch_types=[
            pltpu.VMEM((_C,), jnp.int32),
            pltpu.VMEM((_C,), jnp.int32),
            pltpu.VMEM((_C, _DS), jnp.float32),
            pltpu.VMEM((_C,), jnp.int32),
            pltpu.VMEM((_C,), jnp.int32),
            pltpu.VMEM((_C, _DS), jnp.float32),
            pltpu.VMEM_SHARED((_R, _DS), jnp.float32),
            pltpu.SemaphoreType.DMA,
            pltpu.SemaphoreType.DMA,
            pltpu.SemaphoreType.DMA,
            pltpu.SemaphoreType.DMA,
        ],
    )
    agg, cnt = f(x14, src, dst, zer, ones)
    return agg.reshape(12, _NS, _R, _DS), cnt


def _fuse_weights_kernel(wl_ref, wr_ref, wp_ref, bl_ref, bp_ref,
                         wlp_ref, wrp_ref, cv_ref):
    wp = wp_ref[0]
    wlp_ref[0] = jnp.dot(wp, wl_ref[0], preferred_element_type=jnp.float32)
    wrp_ref[0] = jnp.dot(wp, wr_ref[0], preferred_element_type=jnp.float32)
    cv_ref[0] = bp_ref[0] + lax.dot_general(
        bl_ref[0], wp, (((1,), (1,)), ((), ())),
        preferred_element_type=jnp.float32)


def _main_kernel(x_ref, agg_ref, cnt_ref, wlp_ref, wrp_ref, cv_ref, out_ref):
    for i in range(7):
        base = lax.dot_general(x_ref[i], wrp_ref[i], (((1,), (1,)), ((), ())),
                               preferred_element_type=jnp.float32)
        base = base + cv_ref[i]
        acc = None
        for p, (ii, r) in enumerate(_PAIRS):
            if ii != i:
                continue
            agg = jnp.concatenate([agg_ref[p, s] for s in range(_NS)],
                                  axis=-1)
            cnt = cnt_ref[r, :, 0]
            rec = 1.0 / jnp.maximum(cnt, 1.0)
            logits = lax.dot_general(agg * rec[:, None], wlp_ref[i],
                                     (((1,), (1,)), ((), ())),
                                     preferred_element_type=jnp.float32)
            logits = logits + base
            m = jnp.max(logits, axis=-1, keepdims=True)
            e = jnp.exp(logits - m)
            sm = e / jnp.sum(e, axis=-1, keepdims=True)
            acc = sm if acc is None else acc + sm
        out_ref[i] = acc * (1.0 / _NREL[i])


def kernel(x_Path, x_DNS_Host, x_Package_Name, x_IP, x_Hostnames, x_Command,
           x_Port, Wl, bl, Wr, Wp, bp, ei_accesses, ei_resolves, ei_installs,
           ei_listens, ei_maps, ei_writes):
    xs = [x_Path, x_DNS_Host, x_Package_Name, x_IP, x_Hostnames, x_Command,
          x_Port]
    eis = [ei_accesses, ei_resolves, ei_installs, ei_listens, ei_maps,
           ei_writes]

    x7 = jnp.stack(xs, axis=0)
    x14 = x7.reshape(7, _N, _NS, _DS).transpose(0, 2, 1, 3).reshape(
        7 * _NS, _N, _DS)

    mesh = plsc.VectorSubcoreMesh(core_axis_name="c", subcore_axis_name="s")
    n_sub = mesh.num_subcores
    gran = n_sub * _C * 2
    e_pad = ((_E + gran - 1) // gran) * gran
    pad = e_pad - _E
    src = jnp.stack([jnp.concatenate(
        [ei[0], jnp.zeros((pad,), jnp.int32)]) for ei in eis])
    dst = jnp.stack([jnp.concatenate(
        [ei[1], jnp.full((pad,), _PAD_ROW, jnp.int32)]) for ei in eis])

    agg, cnt = _sc_segment_sums(x14, src, dst, e_pad)

    wlp, wrp, cv = pl.pallas_call(
        _fuse_weights_kernel,
        grid=(7,),
        in_specs=[
            pl.BlockSpec((1, _D, _D), lambda i: (i, 0, 0)),
            pl.BlockSpec((1, _D, _D), lambda i: (i, 0, 0)),
            pl.BlockSpec((1, _D, _D), lambda i: (i, 0, 0)),
            pl.BlockSpec((1, 1, _D), lambda i: (i, 0, 0)),
            pl.BlockSpec((1, 1, _D), lambda i: (i, 0, 0)),
        ],
        out_specs=[
            pl.BlockSpec((1, _D, _D), lambda i: (i, 0, 0)),
            pl.BlockSpec((1, _D, _D), lambda i: (i, 0, 0)),
            pl.BlockSpec((1, 1, _D), lambda i: (i, 0, 0)),
        ],
        out_shape=[
            jax.ShapeDtypeStruct((7, _D, _D), jnp.float32),
            jax.ShapeDtypeStruct((7, _D, _D), jnp.float32),
            jax.ShapeDtypeStruct((7, 1, _D), jnp.float32),
        ],
    )(Wl, Wr, Wp, bl.reshape(7, 1, _D), bp.reshape(7, 1, _D))

    bn = 400
    out = pl.pallas_call(
        _main_kernel,
        grid=(_N // bn,),
        in_specs=[
            pl.BlockSpec((7, bn, _D), lambda t: (0, t, 0)),
            pl.BlockSpec((12, _NS, bn, _DS), lambda t: (0, 0, t, 0)),
            pl.BlockSpec((6, bn, _DS), lambda t: (0, t, 0)),
            pl.BlockSpec((7, _D, _D), lambda t: (0, 0, 0)),
            pl.BlockSpec((7, _D, _D), lambda t: (0, 0, 0)),
            pl.BlockSpec((7, 1, _D), lambda t: (0, 0, 0)),
        ],
        out_specs=pl.BlockSpec((7, bn, _D), lambda t: (0, t, 0)),
        out_shape=jax.ShapeDtypeStruct((7, _N, _D), jnp.float32),
    )(x7, agg, cnt, wlp, wrp, cv)
    return out


# speedup vs baseline: 1.0710x; 1.0710x over previous
import functools
from typing import Sequence

import jax
import jax.numpy as jnp
from jax import lax
from jax.experimental import pallas as pl
from jax.experimental.pallas import tpu as pltpu
from jax.experimental.pallas import tpu_sc as plsc

_NODE_TYPES = ['Path', 'DNS_Host', 'Package_Name', 'IP', 'Hostnames', 'Command', 'Port']
_EDGE_TYPES = [('Path', 'accesses', 'DNS_Host'), ('DNS_Host', 'resolves', 'IP'),
               ('Package_Name', 'installs', 'Path'), ('IP', 'listens', 'Port'),
               ('Hostnames', 'maps', 'IP'), ('Command', 'writes', 'Path')]
_N, _E, _D = 10000, 50000, 256
_NS = 2
_DS = _D // _NS
_R = 10112
_PAD_ROW = _N
_C = 128

_PAIRS = [(0, 0), (0, 2), (0, 5), (1, 0), (1, 1), (2, 2),
          (3, 1), (3, 3), (3, 4), (4, 4), (5, 5), (6, 3)]
_NREL = [3, 2, 1, 3, 1, 1, 1]
_CNT_PAIR = {0: 0, 2: 1, 5: 2, 1: 4, 3: 7, 4: 8}


def _sc_segment_sums(x14, src, dst, e_pad):
    mesh = plsc.VectorSubcoreMesh(core_axis_name="c", subcore_axis_name="s")
    n_cores = mesh.num_cores
    n_sub = mesh.num_subcores
    ep_sub = e_pad // n_sub
    n_chunks = ep_sub // _C
    assert n_chunks % 2 == 0 and n_chunks >= 4
    slab = _R // n_sub

    tasks = []
    for p, (i, r) in enumerate(_PAIRS):
        for h in range(_NS):
            tasks.append(('agg', i * _NS + h, r, p * _NS + h))
    for r in range(6):
        tasks.append(('cnt', None, r, r))
    n_slots = (len(tasks) + n_cores - 1) // n_cores
    by_core = [[None] * n_slots for _ in range(n_cores)]
    for j, t in enumerate(tasks):
        by_core[j % n_cores][j // n_cores] = t

    def body(x_hbm, src_hbm, dst_hbm, zer_hbm, ones_hbm, agg_hbm, cnt_hbm,
             src_a, dst_a, rows_a, src_b, dst_b, rows_b, acc, sem_a, sem_b):
        cid = lax.axis_index("c")
        sid = lax.axis_index("s")
        base_e = sid * ep_sub
        base_r = sid * slab

        def _gather(q, r, k, src_v, rows_v, sem):
            off = base_e + k * _C
            pltpu.sync_copy(src_hbm.at[r, pl.ds(off, _C)], src_v)
            return pltpu.async_copy(x_hbm.at[q].at[src_v], rows_v, sem)

        def _load_dst(r, k, dst_v):
            off = base_e + k * _C
            pltpu.sync_copy(dst_hbm.at[r, pl.ds(off, _C)], dst_v)

        def _scatter(r, k, dst_v, rows_v):
            _load_dst(r, k, dst_v)
            pltpu.sync_copy(rows_v, acc.at[dst_v], add=True)

        for s in range(n_slots):
            for c in range(n_cores):
                task = by_core[c][s]
                if task is None:
                    continue
                kind = task[0]

                @pl.when(cid == c)
                def _zero():
                    pltpu.sync_copy(zer_hbm, acc.at[pl.ds(base_r, slab)])
                    if kind == 'cnt':
                        pltpu.sync_copy(ones_hbm, rows_a)

            plsc.subcore_barrier()

            for c in range(n_cores):
                task = by_core[c][s]
                if task is None:
                    continue
                kind, q, r, g = task

                @pl.when(cid == c)
                def _accumulate():
                    if kind == 'cnt':
                        def _chunk(k, carry):
                            _scatter(r, k, dst_a, rows_a)
                            return carry

                        lax.fori_loop(0, n_chunks, _chunk, 0)
                    else:
                        ga = _gather(q, r, 0, src_a, rows_a, sem_a)

                        def _chunk2(k2, carry):
                            k = 2 * k2
                            gb = _gather(q, r, k + 1, src_b, rows_b, sem_b)
                            pltpu.make_async_copy(
                                x_hbm.at[q].at[src_a], rows_a, sem_a).wait()
                            _scatter(r, k, dst_a, rows_a)
                            ga2 = _gather(q, r, k + 2, src_a, rows_a, sem_a)
                            gb.wait()
                            _scatter(r, k + 1, dst_b, rows_b)
                            return carry

                        lax.fori_loop(0, n_chunks // 2 - 1, _chunk2, 0)
                        k_last = n_chunks - 2
                        gb = _gather(q, r, k_last + 1, src_b, rows_b, sem_b)
                        pltpu.make_async_copy(
                            x_hbm.at[q].at[src_a], rows_a, sem_a).wait()
                        _scatter(r, k_last, dst_a, rows_a)
                        gb.wait()
                        _scatter(r, k_last + 1, dst_b, rows_b)

            plsc.subcore_barrier()

            for c in range(n_cores):
                task = by_core[c][s]
                if task is None:
                    continue
                kind, q, r, g = task

                @pl.when(cid == c)
                def _drain():
                    out = agg_hbm if kind == 'agg' else cnt_hbm
                    pltpu.sync_copy(acc.at[pl.ds(base_r, slab)],
                                    out.at[g, pl.ds(base_r, slab)])

    zer = jnp.zeros((slab, _DS), jnp.float32)
    ones = jnp.ones((_C, _DS), jnp.float32)

    f = pl.kernel(
        body,
        out_type=(jax.ShapeDtypeStruct((12 * _NS, _R, _DS), jnp.float32),
                  jax.ShapeDtypeStruct((6, _R, _DS), jnp.float32)),
        mesh=mesh,
        scratch_types=[
            pltpu.VMEM((_C,), jnp.int32),
            pltpu.VMEM((_C,), jnp.int32),
            pltpu.VMEM((_C, _DS), jnp.float32),
            pltpu.VMEM((_C,), jnp.int32),
            pltpu.VMEM((_C,), jnp.int32),
            pltpu.VMEM((_C, _DS), jnp.float32),
            pltpu.VMEM_SHARED((_R, _DS), jnp.float32),
            pltpu.SemaphoreType.DMA,
            pltpu.SemaphoreType.DMA,
        ],
    )
    agg, cnt = f(x14, src, dst, zer, ones)
    return agg.reshape(12, _NS, _R, _DS), cnt


def _fuse_weights_kernel(wl_ref, wr_ref, wp_ref, bl_ref, bp_ref,
                         wlp_ref, wrp_ref, cv_ref):
    wp = wp_ref[0]
    wlp_ref[0] = jnp.dot(wp, wl_ref[0], preferred_element_type=jnp.float32)
    wrp_ref[0] = jnp.dot(wp, wr_ref[0], preferred_element_type=jnp.float32)
    cv_ref[0] = bp_ref[0] + lax.dot_general(
        bl_ref[0], wp, (((1,), (1,)), ((), ())),
        preferred_element_type=jnp.float32)


def _main_kernel(x_ref, agg_ref, cnt_ref, wlp_ref, wrp_ref, cv_ref, out_ref):
    for i in range(7):
        base = lax.dot_general(x_ref[i], wrp_ref[i], (((1,), (1,)), ((), ())),
                               preferred_element_type=jnp.float32)
        base = base + cv_ref[i]
        acc = None
        for p, (ii, r) in enumerate(_PAIRS):
            if ii != i:
                continue
            agg = jnp.concatenate([agg_ref[p, s] for s in range(_NS)],
                                  axis=-1)
            cnt = cnt_ref[r, :, 0]
            rec = 1.0 / jnp.maximum(cnt, 1.0)
            logits = lax.dot_general(agg * rec[:, None], wlp_ref[i],
                                     (((1,), (1,)), ((), ())),
                                     preferred_element_type=jnp.float32)
            logits = logits + base
            m = jnp.max(logits, axis=-1, keepdims=True)
            e = jnp.exp(logits - m)
            sm = e / jnp.sum(e, axis=-1, keepdims=True)
            acc = sm if acc is None else acc + sm
        out_ref[i] = acc * (1.0 / _NREL[i])


def kernel(x_Path, x_DNS_Host, x_Package_Name, x_IP, x_Hostnames, x_Command,
           x_Port, Wl, bl, Wr, Wp, bp, ei_accesses, ei_resolves, ei_installs,
           ei_listens, ei_maps, ei_writes):
    xs = [x_Path, x_DNS_Host, x_Package_Name, x_IP, x_Hostnames, x_Command,
          x_Port]
    eis = [ei_accesses, ei_resolves, ei_installs, ei_listens, ei_maps,
           ei_writes]

    x7 = jnp.stack(xs, axis=0)
    x14 = x7.reshape(7, _N, _NS, _DS).transpose(0, 2, 1, 3).reshape(
        7 * _NS, _N, _DS)

    mesh = plsc.VectorSubcoreMesh(core_axis_name="c", subcore_axis_name="s")
    n_sub = mesh.num_subcores
    gran = n_sub * _C * 2
    e_pad = ((_E + gran - 1) // gran) * gran
    pad = e_pad - _E
    src = jnp.stack([jnp.concatenate(
        [ei[0], jnp.zeros((pad,), jnp.int32)]) for ei in eis])
    dst = jnp.stack([jnp.concatenate(
        [ei[1], jnp.full((pad,), _PAD_ROW, jnp.int32)]) for ei in eis])

    agg, cnt = _sc_segment_sums(x14, src, dst, e_pad)

    wlp, wrp, cv = pl.pallas_call(
        _fuse_weights_kernel,
        grid=(7,),
        in_specs=[
            pl.BlockSpec((1, _D, _D), lambda i: (i, 0, 0)),
            pl.BlockSpec((1, _D, _D), lambda i: (i, 0, 0)),
            pl.BlockSpec((1, _D, _D), lambda i: (i, 0, 0)),
            pl.BlockSpec((1, 1, _D), lambda i: (i, 0, 0)),
            pl.BlockSpec((1, 1, _D), lambda i: (i, 0, 0)),
        ],
        out_specs=[
            pl.BlockSpec((1, _D, _D), lambda i: (i, 0, 0)),
            pl.BlockSpec((1, _D, _D), lambda i: (i, 0, 0)),
            pl.BlockSpec((1, 1, _D), lambda i: (i, 0, 0)),
        ],
        out_shape=[
            jax.ShapeDtypeStruct((7, _D, _D), jnp.float32),
            jax.ShapeDtypeStruct((7, _D, _D), jnp.float32),
            jax.ShapeDtypeStruct((7, 1, _D), jnp.float32),
        ],
    )(Wl, Wr, Wp, bl.reshape(7, 1, _D), bp.reshape(7, 1, _D))

    bn = 400
    out = pl.pallas_call(
        _main_kernel,
        grid=(_N // bn,),
        in_specs=[
            pl.BlockSpec((7, bn, _D), lambda t: (0, t, 0)),
            pl.BlockSpec((12, _NS, bn, _DS), lambda t: (0, 0, t, 0)),
            pl.BlockSpec((6, bn, _DS), lambda t: (0, t, 0)),
            pl.BlockSpec((7, _D, _D), lambda t: (0, 0, 0)),
            pl.BlockSpec((7, _D, _D), lambda t: (0, 0, 0)),
            pl.BlockSpec((7, 1, _D), lambda t: (0, 0, 0)),
        ],
        out_specs=pl.BlockSpec((7, bn, _D), lambda t: (0, t, 0)),
        out_shape=jax.ShapeDtypeStruct((7, _N, _D), jnp.float32),
    )(x7, agg, cnt, wlp, wrp, cv)
    return out
